# Initial kernel scaffold; baseline (speedup 1.0000x reference)
import jax
import jax.numpy as jnp
from jax.experimental import pallas as pl


def kernel(x, edge_index0, edge_weight0, edge_index1, edge_weight1, pos_edges, neg_edges, W1_0, b1_0, W2_0, b2_0, W1_1, b1_1, W2_1, b2_1, Wp1, bp1, Wp2, bp2):
    raise NotImplementedError("write your pallas kernel here")


# speedup vs baseline: 4.6674x; 4.6674x over previous
import functools

import jax
import jax.numpy as jnp
from jax import lax
from jax.experimental import pallas as pl
from jax.experimental.pallas import tpu as pltpu
from jax.experimental.pallas import tpu_sc as plsc

_NC = 2
_NS = 16
_L = 16
_NW = _NC * _NS

_GATHER_DNUMS = lax.GatherDimensionNumbers(
    offset_dims=(), collapsed_slice_dims=(0,), start_index_map=(0,))


def _lane_bcast(vec, lane):
    idx = jnp.full((_L, 1), lane, jnp.int32)
    return lax.gather(vec, idx, _GATHER_DNUMS, slice_sizes=(1,),
                      mode=lax.GatherScatterMode.PROMISE_IN_BOUNDS)


def _sc_agg(table, src, dst, w):
    n, d = table.shape
    e = src.shape[0]
    k = 80
    tile_e = e // _NW
    ch = tile_e // k
    zr = k
    g_rows = n // zr
    r_trips = (g_rows + _NS - 1) // _NS
    nvr = d // _L

    mesh = plsc.VectorSubcoreMesh(core_axis_name="c", subcore_axis_name="s")

    @functools.partial(
        pl.kernel,
        mesh=mesh,
        out_type=jax.ShapeDtypeStruct((_NC, n, d), jnp.float32),
        scratch_types=[
            pltpu.VMEM((tile_e,), jnp.int32),
            pltpu.VMEM((tile_e,), jnp.float32),
            pltpu.VMEM((k,), jnp.int32),
            pltpu.VMEM((k, d), jnp.float32),
            pltpu.VMEM_SHARED((n, d), jnp.float32),
        ],
    )
    def agg(table_h, src_h, dst_h, w_h, out_h, src_v, w_v, dstc_v, rows_v,
            acc):
        c = lax.axis_index("c")
        s = lax.axis_index("s")
        wid = c * _NS + s

        zv = jnp.zeros((_L,), jnp.float32)

        def zrow(r, carry):
            for f in range(nvr):
                rows_v[r, pl.ds(f * _L, _L)] = zv
            return carry

        lax.fori_loop(0, zr, zrow, 0)

        def zchunk(t, carry):
            g = s + t * _NS

            @pl.when(g < g_rows)
            def _():
                pltpu.sync_copy(rows_v, acc.at[pl.ds(g * zr, zr)])

            return carry

        lax.fori_loop(0, r_trips, zchunk, 0)
        plsc.subcore_barrier()

        base = wid * tile_e
        pltpu.sync_copy(src_h.at[pl.ds(base, tile_e)], src_v)
        pltpu.sync_copy(w_h.at[pl.ds(base, tile_e)], w_v)

        def chunk(j, carry):
            cb = j * k
            pltpu.sync_copy(dst_h.at[pl.ds(base + cb, k)], dstc_v)
            pltpu.sync_copy(table_h.at[src_v.at[pl.ds(cb, k)]], rows_v)
            for q in range(k // _L):
                wrow = w_v[pl.ds(cb + q * _L, _L)]
                for r in range(_L):
                    i = q * _L + r
                    wv = _lane_bcast(wrow, r)
                    for f in range(nvr):
                        sl = (i, pl.ds(f * _L, _L))
                        rows_v[sl] = rows_v[sl] * wv
            pltpu.sync_copy(rows_v, acc.at[dstc_v], add=True)
            return carry

        lax.fori_loop(0, ch, chunk, 0)
        plsc.subcore_barrier()

        def dchunk(t, carry):
            g = s + t * _NS

            @pl.when(g < g_rows)
            def _():
                r0 = g * zr
                pltpu.sync_copy(acc.at[pl.ds(r0, zr)],
                                out_h.at[c, pl.ds(r0, zr)])

            return carry

        lax.fori_loop(0, r_trips, dchunk, 0)

    return agg(table, src, dst, w)


def _sc_gather2(table, a_idx, b_idx):
    n, d = table.shape
    b = a_idx.shape[0]
    k = 80
    g_total = b // k
    trips = (g_total + _NW - 1) // _NW

    mesh = plsc.VectorSubcoreMesh(core_axis_name="c", subcore_axis_name="s")

    @functools.partial(
        pl.kernel,
        mesh=mesh,
        out_type=jax.ShapeDtypeStruct((2, b, d), jnp.float32),
        scratch_types=[
            pltpu.VMEM((k,), jnp.int32),
            pltpu.VMEM((k, d), jnp.float32),
        ],
    )
    def gat(table_h, aidx_h, bidx_h, out_h, idx_v, rows_v):
        c = lax.axis_index("c")
        s = lax.axis_index("s")
        wid = c * _NS + s

        def chunk(t, carry):
            g = wid + t * _NW

            @pl.when(g < g_total)
            def _():
                cb = g * k
                pltpu.sync_copy(aidx_h.at[pl.ds(cb, k)], idx_v)
                pltpu.sync_copy(table_h.at[idx_v], rows_v)
                pltpu.sync_copy(rows_v, out_h.at[0, pl.ds(cb, k)])
                pltpu.sync_copy(bidx_h.at[pl.ds(cb, k)], idx_v)
                pltpu.sync_copy(table_h.at[idx_v], rows_v)
                pltpu.sync_copy(rows_v, out_h.at[1, pl.ds(cb, k)])

            return carry

        lax.fori_loop(0, trips, chunk, 0)

    return gat(table, a_idx, b_idx)


def _tc_mlp(parts, w1, b1, w2, b2):
    _, n, d = parts.shape
    mh = w1.shape[1]
    h = w2.shape[1]
    bn = 1000

    def body(p_ref, w1_ref, b1_ref, w2_ref, b2_ref, o_ref):
        a = p_ref[0] + p_ref[1]
        hid = jnp.dot(a, w1_ref[...], preferred_element_type=jnp.float32)
        hid = jnp.maximum(hid + b1_ref[...], 0.0)
        out = jnp.dot(hid, w2_ref[...], preferred_element_type=jnp.float32)
        o_ref[...] = jnp.maximum(out + b2_ref[...], 0.0)

    return pl.pallas_call(
        body,
        grid=(n // bn,),
        in_specs=[
            pl.BlockSpec((2, bn, d), lambda i: (0, i, 0)),
            pl.BlockSpec((d, mh), lambda i: (0, 0)),
            pl.BlockSpec((1, mh), lambda i: (0, 0)),
            pl.BlockSpec((mh, h), lambda i: (0, 0)),
            pl.BlockSpec((1, h), lambda i: (0, 0)),
        ],
        out_specs=pl.BlockSpec((bn, h), lambda i: (i, 0)),
        out_shape=jax.ShapeDtypeStruct((n, h), jnp.float32),
    )(parts, w1, b1.reshape(1, mh), w2, b2.reshape(1, h))


def _tc_predict(pairs, wp1, bp1, wp2, bp2):
    _, b, d = pairs.shape
    hh = wp1.shape[1]
    fd = wp2.shape[1]
    bn = 1000

    def body(p_ref, w1_ref, b1_ref, w2_ref, b2_ref, o_ref):
        a = p_ref[0] * p_ref[1]
        z = jnp.dot(a, w1_ref[...], preferred_element_type=jnp.float32)
        z = jnp.maximum(z + b1_ref[...], 0.0)
        out = jnp.dot(z, w2_ref[...], preferred_element_type=jnp.float32)
        o_ref[...] = out + b2_ref[...]

    return pl.pallas_call(
        body,
        grid=(b // bn,),
        in_specs=[
            pl.BlockSpec((2, bn, d), lambda i: (0, i, 0)),
            pl.BlockSpec((d, hh), lambda i: (0, 0)),
            pl.BlockSpec((1, hh), lambda i: (0, 0)),
            pl.BlockSpec((hh, fd), lambda i: (0, 0)),
            pl.BlockSpec((1, fd), lambda i: (0, 0)),
        ],
        out_specs=pl.BlockSpec((bn, fd), lambda i: (i, 0)),
        out_shape=jax.ShapeDtypeStruct((b, fd), jnp.float32),
    )(pairs, wp1, bp1.reshape(1, hh), wp2, bp2.reshape(1, fd))


def kernel(x, edge_index0, edge_weight0, edge_index1, edge_weight1,
           pos_edges, neg_edges, W1_0, b1_0, W2_0, b2_0, W1_1, b1_1,
           W2_1, b2_1, Wp1, bp1, Wp2, bp2):
    p = _sc_agg(x, edge_index0[0], edge_index0[1], edge_weight0)
    h = _tc_mlp(p, W1_0, b1_0, W2_0, b2_0)
    p = _sc_agg(h, edge_index1[0], edge_index1[1], edge_weight1)
    h = _tc_mlp(p, W1_1, b1_1, W2_1, b2_1)

    a_idx = jnp.concatenate([pos_edges[0], neg_edges[0]])
    b_idx = jnp.concatenate([pos_edges[1], neg_edges[1]])
    pairs = _sc_gather2(h, a_idx, b_idx)
    pred = _tc_predict(pairs, Wp1, bp1, Wp2, bp2)

    p_cnt = pos_edges.shape[1]
    return pred[:p_cnt], pred[p_cnt:], h
